# baseline (speedup 1.0000x reference)
import functools

import jax
import jax.numpy as jnp
from jax import lax
from jax.experimental import pallas as pl
from jax.experimental.pallas import tpu as pltpu
from jax.experimental.pallas import tpu_sc as plsc

B, C, L, D = 64, 8, 576, 192
KEEP = 144
NW = 32
B_PER_W = B // NW
LCH = L // 16

RANK_BLK = 8


def _rank_body(noise_ref, rank_ref):
    li = lax.broadcasted_iota(jnp.int32, (L, L), 0)
    ji = lax.broadcasted_iota(jnp.int32, (L, L), 1)
    tie = ji < li
    for i in range(RANK_BLK):
        row = noise_ref[i, :]
        a = row[:, None]
        bt = row[None, :]
        cmp = (bt < a) | ((bt == a) & tie)
        rank_ref[i, :] = jnp.sum(cmp.astype(jnp.int32), axis=1)


def _compute_ranks(noise):
    return pl.pallas_call(
        _rank_body,
        grid=(B // RANK_BLK,),
        in_specs=[pl.BlockSpec((RANK_BLK, L), lambda b: (b, 0))],
        out_specs=pl.BlockSpec((RANK_BLK, L), lambda b: (b, 0)),
        out_shape=jax.ShapeDtypeStruct((B, L), jnp.int32),
    )(noise)


def _gather_body(rank_ref, x_ref, xk_ref):
    rank_row = rank_ref[0, 0, :]
    kk = lax.broadcasted_iota(jnp.int32, (KEEP, L), 0)
    for i in range(4):
        p = (rank_ref[i, 0, :][None, :] == kk).astype(jnp.bfloat16)
        for c in range(C):
            xb = x_ref[i, c]
            x_hi = xb.astype(jnp.bfloat16)
            x_lo = (xb - x_hi.astype(jnp.float32)).astype(jnp.bfloat16)
            hi = jnp.dot(p, x_hi, preferred_element_type=jnp.float32)
            lo = jnp.dot(p, x_lo, preferred_element_type=jnp.float32)
            xk_ref[i, c] = hi + lo


def _gather_kept(rank, x):
    return pl.pallas_call(
        _gather_body,
        grid=(B // 4,),
        in_specs=[
            pl.BlockSpec((4, 1, L), lambda b: (b, 0, 0)),
            pl.BlockSpec((4, C, L, D), lambda b: (b, 0, 0, 0)),
        ],
        out_specs=pl.BlockSpec((4, C, KEEP, D), lambda b: (b, 0, 0, 0)),
        out_shape=jax.ShapeDtypeStruct((B, C, KEEP, D), jnp.float32),
    )(rank.reshape(B, 1, L), x)


def _sc_body(rank_ref, idr_ref, mask_ref, idk_ref, rank_row, keep, mrow):
    cid = lax.axis_index("c")
    sid = lax.axis_index("s")
    wid = sid * 2 + cid
    for i in range(B_PER_W):
        b = wid * B_PER_W + i
        pltpu.sync_copy(rank_ref.at[pl.ds(b * L, L)], rank_row)
        for k in range(LCH):
            r = rank_row[pl.ds(k * 16, 16)]
            lvec = lax.iota(jnp.int32, 16) + (k * 16)
            m = r < KEEP
            idx = jnp.where(m, r, 0)
            plsc.store_scatter(keep, [idx], lvec, mask=m)
            mrow[pl.ds(k * 16, 16)] = jnp.where(
                m, jnp.float32(0.0), jnp.float32(1.0))
        for c in range(C):
            bc = b * C + c
            pltpu.sync_copy(rank_row, idr_ref.at[pl.ds(bc * L, L)])
            pltpu.sync_copy(mrow, mask_ref.at[pl.ds(bc * L, L)])
            pltpu.sync_copy(keep, idk_ref.at[pl.ds(bc * KEEP, KEEP)])


@functools.cache
def _sc_perm_outputs():
    return pl.kernel(
        _sc_body,
        out_type=(
            jax.ShapeDtypeStruct((B * C * L,), jnp.int32),
            jax.ShapeDtypeStruct((B * C * L,), jnp.float32),
            jax.ShapeDtypeStruct((B * C * KEEP,), jnp.int32),
        ),
        mesh=plsc.VectorSubcoreMesh(core_axis_name="c", subcore_axis_name="s"),
        scratch_types=[
            pltpu.VMEM((L,), jnp.int32),
            pltpu.VMEM((KEEP,), jnp.int32),
            pltpu.VMEM((L,), jnp.float32),
        ],
        compiler_params=pltpu.CompilerParams(needs_layout_passes=False),
    )


def kernel(x):
    assert x.shape == (B, C, L, D), x.shape
    noise = jax.random.uniform(jax.random.key(1), (B, L), dtype=jnp.float32)
    rank = _compute_ranks(noise)
    xk = _gather_kept(rank, x)
    idr, mask, idk = _sc_perm_outputs()(rank.reshape(B * L))
    return (xk, idr.reshape(B, C, L),
            mask.reshape(B, C, L), idk.reshape(B, C, KEEP))


# speedup vs baseline: 1.1207x; 1.1207x over previous
import functools

import jax
import jax.numpy as jnp
from jax import lax
from jax.experimental import pallas as pl
from jax.experimental.pallas import tpu as pltpu
from jax.experimental.pallas import tpu_sc as plsc

B, C, L, D = 64, 8, 576, 192
KEEP = 144
NW = 32
B_PER_W = B // NW
LCH = L // 16

RANK_BLK = 8


def _rank_body(noise_ref, rank_ref):
    li = lax.broadcasted_iota(jnp.int32, (L, L), 0)
    ji = lax.broadcasted_iota(jnp.int32, (L, L), 1)
    tie = ji < li
    for i in range(RANK_BLK):
        row = noise_ref[i, :]
        a = row[:, None]
        bt = row[None, :]
        cmp = (bt < a) | ((bt == a) & tie)
        rank_ref[i, :] = jnp.sum(cmp.astype(jnp.int32), axis=1)


def _compute_ranks(noise):
    return pl.pallas_call(
        _rank_body,
        grid=(B // RANK_BLK,),
        in_specs=[pl.BlockSpec((RANK_BLK, L), lambda b: (b, 0))],
        out_specs=pl.BlockSpec((RANK_BLK, L), lambda b: (b, 0)),
        out_shape=jax.ShapeDtypeStruct((B, L), jnp.int32),
    )(noise)


def _gather_body(rank_ref, x_ref, xk_ref):
    rank_row = rank_ref[0, 0, :]
    kk = lax.broadcasted_iota(jnp.int32, (KEEP, L), 0)
    for i in range(4):
        p = (rank_ref[i, 0, :][None, :] == kk).astype(jnp.bfloat16)
        for c in range(C):
            xb = x_ref[i, c]
            x_hi = xb.astype(jnp.bfloat16)
            x_lo = (xb - x_hi.astype(jnp.float32)).astype(jnp.bfloat16)
            hi = jnp.dot(p, x_hi, preferred_element_type=jnp.float32)
            lo = jnp.dot(p, x_lo, preferred_element_type=jnp.float32)
            xk_ref[i, c] = hi + lo


def _gather_kept(rank, x):
    return pl.pallas_call(
        _gather_body,
        grid=(B // 4,),
        in_specs=[
            pl.BlockSpec((4, 1, L), lambda b: (b, 0, 0)),
            pl.BlockSpec((4, C, L, D), lambda b: (b, 0, 0, 0)),
        ],
        out_specs=pl.BlockSpec((4, C, KEEP, D), lambda b: (b, 0, 0, 0)),
        out_shape=jax.ShapeDtypeStruct((B, C, KEEP, D), jnp.float32),
    )(rank.reshape(B, 1, L), x)


def _sc_body(rank_ref, x_ref, xk_ref, idr_ref, mask_ref, idk_ref,
             rank_row, keep, mrow, rows, sem):
    cid = lax.axis_index("c")
    sid = lax.axis_index("s")
    wid = sid * 2 + cid
    lane = lax.iota(jnp.int32, 16)
    for i in range(B_PER_W):
        b = wid * B_PER_W + i
        pltpu.sync_copy(rank_ref.at[pl.ds(b * L, L)], rank_row)
        for k in range(LCH):
            r = rank_row[pl.ds(k * 16, 16)]
            lvec = lax.iota(jnp.int32, 16) + (k * 16)
            m = r < KEEP
            idx = jnp.where(m, r, 0)
            plsc.store_scatter(keep, [idx], lvec, mask=m)
            mrow[pl.ds(k * 16, 16)] = jnp.where(
                m, jnp.float32(0.0), jnp.float32(1.0))

        def per_c(c, carry, b=b):
            bc = b * C + c
            pltpu.sync_copy(
                rank_row, idr_ref.at[pl.ds(pl.multiple_of(bc * L, 8), L)])
            pltpu.sync_copy(
                mrow, mask_ref.at[pl.ds(pl.multiple_of(bc * L, 8), L)])
            pltpu.sync_copy(
                keep, idk_ref.at[pl.ds(pl.multiple_of(bc * KEEP, 8), KEEP)])
            for kc in range(KEEP // 16):
                kchunk = keep[pl.ds(kc * 16, 16)]

                def issue(j, c2, kchunk=kchunk, kc=kc):
                    l_idx = jnp.sum(jnp.where(lane == j, kchunk, 0))
                    pltpu.async_copy(
                        x_ref.at[b, c, pl.ds(l_idx, 1), :],
                        rows.at[pl.ds(kc * 16 + j, 1), :], sem)
                    return c2

                lax.fori_loop(0, 16, issue, 0)

            def drain(k, c2):
                pltpu.make_async_copy(
                    x_ref.at[0, 0, pl.ds(0, 1), :],
                    rows.at[pl.ds(k, 1), :], sem).wait()
                return c2

            lax.fori_loop(0, KEEP, drain, 0)
            pltpu.sync_copy(rows, xk_ref.at[b, c])
            return carry

        lax.fori_loop(0, C, per_c, 0)


@functools.cache
def _sc_dropout():
    return pl.kernel(
        _sc_body,
        out_type=(
            jax.ShapeDtypeStruct((B, C, KEEP, D), jnp.float32),
            jax.ShapeDtypeStruct((B * C * L,), jnp.int32),
            jax.ShapeDtypeStruct((B * C * L,), jnp.float32),
            jax.ShapeDtypeStruct((B * C * KEEP,), jnp.int32),
        ),
        mesh=plsc.VectorSubcoreMesh(core_axis_name="c", subcore_axis_name="s"),
        scratch_types=[
            pltpu.VMEM((L,), jnp.int32),
            pltpu.VMEM((KEEP,), jnp.int32),
            pltpu.VMEM((L,), jnp.float32),
            pltpu.VMEM((KEEP, D), jnp.float32),
            pltpu.SemaphoreType.DMA,
        ],
        compiler_params=pltpu.CompilerParams(needs_layout_passes=False),
    )


def kernel(x):
    assert x.shape == (B, C, L, D), x.shape
    noise = jax.random.uniform(jax.random.key(1), (B, L), dtype=jnp.float32)
    rank = _compute_ranks(noise)
    xk, idr, mask, idk = _sc_dropout()(rank.reshape(B * L), x)
    return (xk, idr.reshape(B, C, L),
            mask.reshape(B, C, L), idk.reshape(B, C, KEEP))
